# baseline (device time: 401806 ns/iter reference)
import jax
import jax.numpy as jnp
from jax import lax
from jax.experimental import pallas as pl
from jax.experimental.pallas import tpu as pltpu

N_DEV = 8

_sem_signal = getattr(pl, "semaphore_signal", None) or pltpu.semaphore_signal
_sem_wait = getattr(pl, "semaphore_wait", None) or pltpu.semaphore_wait
_DevIdType = getattr(pl, "DeviceIdType", None) or pltpu.DeviceIdType
_CompilerParams = getattr(pltpu, "CompilerParams", None) or pltpu.TPUCompilerParams


def kernel(x, w_mat, scale_x, scale_w):
    M, k_per = x.shape
    k_per2, N = w_mat.shape
    assert k_per == k_per2
    m_per = M // N_DEV

    def body(x_ref, w_ref, sx_ref, sw_ref, out_ref,
             xg_ref, wg_ref,
             x_send_sems, x_recv_sems, w_send_sems, w_recv_sems,
             credit_sem):
        my = lax.axis_index("i")
        left = lax.rem(my + N_DEV - 1, N_DEV)
        right = lax.rem(my + 1, N_DEV)

        barrier = pltpu.get_barrier_semaphore()
        for j in range(1, N_DEV):
            _sem_signal(barrier, inc=1,
                        device_id=(lax.rem(my + j, N_DEV),),
                        device_id_type=_DevIdType.MESH)
        _sem_wait(barrier, N_DEV - 1)

        x_sends = []
        for j in range(1, N_DEV):
            t = lax.rem(my + j, N_DEV)
            rd = pltpu.make_async_remote_copy(
                src_ref=x_ref.at[pl.ds(t * m_per, m_per), :],
                dst_ref=xg_ref.at[my],
                send_sem=x_send_sems.at[t],
                recv_sem=x_recv_sems.at[my],
                device_id=(t,),
                device_id_type=_DevIdType.MESH,
            )
            rd.start()
            x_sends.append(rd)

        x_own = x_ref[pl.ds(my * m_per, m_per), :]
        out_ref[...] = jnp.dot(
            x_own.astype(jnp.bfloat16), w_ref[...].astype(jnp.bfloat16),
            preferred_element_type=jnp.float32)

        wg_ref[0] = w_ref[...]
        for h in range(N_DEV - 1):
            s_slot = h % 2
            r_slot = (h + 1) % 2
            if h >= 1:
                _sem_wait(credit_sem.at[r_slot], 1)
            rd = pltpu.make_async_remote_copy(
                src_ref=wg_ref.at[s_slot],
                dst_ref=wg_ref.at[r_slot],
                send_sem=w_send_sems.at[s_slot],
                recv_sem=w_recv_sems.at[r_slot],
                device_id=(right,),
                device_id_type=_DevIdType.MESH,
            )
            rd.start()
            rd.wait()
            if h < N_DEV - 2:
                _sem_signal(credit_sem.at[s_slot], inc=1,
                            device_id=(left,),
                            device_id_type=_DevIdType.MESH)

            c = lax.rem(my + N_DEV - h - 1, N_DEV)
            xr = pltpu.make_async_remote_copy(
                src_ref=xg_ref.at[c],
                dst_ref=xg_ref.at[c],
                send_sem=x_send_sems.at[c],
                recv_sem=x_recv_sems.at[c],
                device_id=(left,),
                device_id_type=_DevIdType.MESH,
            )
            xr.wait_recv()
            out_ref[...] += jnp.dot(
                xg_ref[c].astype(jnp.bfloat16),
                wg_ref[r_slot].astype(jnp.bfloat16),
                preferred_element_type=jnp.float32)

        for rd in x_sends:
            rd.wait_send()

        out_ref[...] = out_ref[...] * (sx_ref[0] * sw_ref[0])

    return pl.pallas_call(
        body,
        out_shape=jax.ShapeDtypeStruct((m_per, N), jnp.float32),
        in_specs=[
            pl.BlockSpec(memory_space=pltpu.VMEM),
            pl.BlockSpec(memory_space=pltpu.VMEM),
            pl.BlockSpec(memory_space=pltpu.SMEM),
            pl.BlockSpec(memory_space=pltpu.SMEM),
        ],
        out_specs=pl.BlockSpec(memory_space=pltpu.VMEM),
        scratch_shapes=[
            pltpu.VMEM((N_DEV, m_per, k_per), jnp.int8),
            pltpu.VMEM((2, k_per, N), jnp.int8),
            pltpu.SemaphoreType.DMA((N_DEV,)),
            pltpu.SemaphoreType.DMA((N_DEV,)),
            pltpu.SemaphoreType.DMA((2,)),
            pltpu.SemaphoreType.DMA((2,)),
            pltpu.SemaphoreType.REGULAR((2,)),
        ],
        compiler_params=_CompilerParams(collective_id=0),
    )(x, w_mat, scale_x, scale_w)


# device time: 218888 ns/iter; 1.8357x vs baseline; 1.8357x over previous
import jax
import jax.numpy as jnp
from jax import lax
from jax.experimental import pallas as pl
from jax.experimental.pallas import tpu as pltpu

N_DEV = 8

_sem_signal = getattr(pl, "semaphore_signal", None) or pltpu.semaphore_signal
_sem_wait = getattr(pl, "semaphore_wait", None) or pltpu.semaphore_wait
_DevIdType = getattr(pl, "DeviceIdType", None) or pltpu.DeviceIdType
_CompilerParams = getattr(pltpu, "CompilerParams", None) or pltpu.TPUCompilerParams


def kernel(x, w_mat, scale_x, scale_w):
    M, k_per = x.shape
    k_per2, N = w_mat.shape
    assert k_per == k_per2
    m_per = M // N_DEV
    N2 = N // 2

    def body(x_ref, w_ref, sx_ref, sw_ref, out_ref,
             xg_ref, wgf_ref, wgb_ref,
             x_send_sems, x_recv_sems,
             f_send_sems, f_recv_sems, b_send_sems, b_recv_sems,
             creditf, creditb):
        my = lax.axis_index("i")
        left = lax.rem(my + N_DEV - 1, N_DEV)
        right = lax.rem(my + 1, N_DEV)

        barrier = pltpu.get_barrier_semaphore()
        for j in range(1, N_DEV):
            _sem_signal(barrier, inc=1,
                        device_id=(lax.rem(my + j, N_DEV),),
                        device_id_type=_DevIdType.MESH)
        _sem_wait(barrier, N_DEV - 1)

        x_sends = []
        for j in range(1, N_DEV):
            t = lax.rem(my + j, N_DEV)
            rd = pltpu.make_async_remote_copy(
                src_ref=x_ref.at[pl.ds(t * m_per, m_per), :],
                dst_ref=xg_ref.at[my],
                send_sem=x_send_sems.at[t],
                recv_sem=x_recv_sems.at[my],
                device_id=(t,),
                device_id_type=_DevIdType.MESH,
            )
            rd.start()
            x_sends.append(rd)

        def xg_wait(c):
            xr = pltpu.make_async_remote_copy(
                src_ref=xg_ref.at[c],
                dst_ref=xg_ref.at[c],
                send_sem=x_send_sems.at[c],
                recv_sem=x_recv_sems.at[c],
                device_id=(left,),
                device_id_type=_DevIdType.MESH,
            )
            xr.wait_recv()

        def send(direction, h):
            if direction == 0:
                src = w_ref.at[:, pl.ds(0, N2)] if h == 0 \
                    else wgf_ref.at[(h - 1) % 2]
                rd = pltpu.make_async_remote_copy(
                    src_ref=src,
                    dst_ref=wgf_ref.at[h % 2],
                    send_sem=f_send_sems.at[h % 2],
                    recv_sem=f_recv_sems.at[h % 2],
                    device_id=(right,),
                    device_id_type=_DevIdType.MESH,
                )
            else:
                src = w_ref.at[:, pl.ds(N2, N2)] if h == 0 \
                    else wgb_ref.at[(h - 1) % 2]
                rd = pltpu.make_async_remote_copy(
                    src_ref=src,
                    dst_ref=wgb_ref.at[h % 2],
                    send_sem=b_send_sems.at[h % 2],
                    recv_sem=b_recv_sems.at[h % 2],
                    device_id=(left,),
                    device_id_type=_DevIdType.MESH,
                )
            rd.start()
            return rd

        bf16 = jnp.bfloat16
        f32 = jnp.float32

        rdf = send(0, 0)
        rdb = send(1, 0)
        x_own = x_ref[pl.ds(my * m_per, m_per), :]
        out_ref[...] = jnp.dot(
            x_own.astype(bf16), w_ref[...].astype(bf16),
            preferred_element_type=f32)
        rdf.wait_send()
        rdb.wait_send()
        rdf.wait_recv()
        rdb.wait_recv()

        for h in range(1, N_DEV - 1):
            if h >= 2:
                _sem_wait(creditf.at[h % 2], 1)
                _sem_wait(creditb.at[h % 2], 1)
            rdf = send(0, h)
            rdb = send(1, h)

            cf = lax.rem(my + N_DEV - h, N_DEV)
            cb = lax.rem(my + h, N_DEV)
            if h <= 3:
                xg_wait(cf)
                xg_wait(cb)
            elif h == 4:
                xg_wait(cb)
            out_ref[:, pl.ds(0, N2)] += jnp.dot(
                xg_ref[cf].astype(bf16), wgf_ref[(h - 1) % 2].astype(bf16),
                preferred_element_type=f32)
            out_ref[:, pl.ds(N2, N2)] += jnp.dot(
                xg_ref[cb].astype(bf16), wgb_ref[(h - 1) % 2].astype(bf16),
                preferred_element_type=f32)

            rdf.wait_send()
            rdb.wait_send()
            if h < N_DEV - 2:
                _sem_signal(creditf.at[(h - 1) % 2], inc=1,
                            device_id=(left,),
                            device_id_type=_DevIdType.MESH)
                _sem_signal(creditb.at[(h - 1) % 2], inc=1,
                            device_id=(right,),
                            device_id_type=_DevIdType.MESH)
            rdf.wait_recv()
            rdb.wait_recv()

        cf = lax.rem(my + 1, N_DEV)
        cb = lax.rem(my + N_DEV - 1, N_DEV)
        out_ref[:, pl.ds(0, N2)] += jnp.dot(
            xg_ref[cf].astype(bf16), wgf_ref[(N_DEV - 2) % 2].astype(bf16),
            preferred_element_type=f32)
        out_ref[:, pl.ds(N2, N2)] += jnp.dot(
            xg_ref[cb].astype(bf16), wgb_ref[(N_DEV - 2) % 2].astype(bf16),
            preferred_element_type=f32)
        for rd in x_sends:
            rd.wait_send()

        out_ref[...] = out_ref[...] * (sx_ref[0] * sw_ref[0])

    return pl.pallas_call(
        body,
        out_shape=jax.ShapeDtypeStruct((m_per, N), jnp.float32),
        in_specs=[
            pl.BlockSpec(memory_space=pltpu.VMEM),
            pl.BlockSpec(memory_space=pltpu.VMEM),
            pl.BlockSpec(memory_space=pltpu.SMEM),
            pl.BlockSpec(memory_space=pltpu.SMEM),
        ],
        out_specs=pl.BlockSpec(memory_space=pltpu.VMEM),
        scratch_shapes=[
            pltpu.VMEM((N_DEV, m_per, k_per), jnp.int8),
            pltpu.VMEM((2, k_per, N2), jnp.int8),
            pltpu.VMEM((2, k_per, N2), jnp.int8),
            pltpu.SemaphoreType.DMA((N_DEV,)),
            pltpu.SemaphoreType.DMA((N_DEV,)),
            pltpu.SemaphoreType.DMA((2,)),
            pltpu.SemaphoreType.DMA((2,)),
            pltpu.SemaphoreType.DMA((2,)),
            pltpu.SemaphoreType.DMA((2,)),
            pltpu.SemaphoreType.REGULAR((2,)),
            pltpu.SemaphoreType.REGULAR((2,)),
        ],
        compiler_params=_CompilerParams(collective_id=0),
    )(x, w_mat, scale_x, scale_w)


# device time: 215122 ns/iter; 1.8678x vs baseline; 1.0175x over previous
import jax
import jax.numpy as jnp
from jax import lax
from jax.experimental import pallas as pl
from jax.experimental.pallas import tpu as pltpu

N_DEV = 8
N_SLOT = 4

_sem_signal = getattr(pl, "semaphore_signal", None) or pltpu.semaphore_signal
_sem_wait = getattr(pl, "semaphore_wait", None) or pltpu.semaphore_wait
_DevIdType = getattr(pl, "DeviceIdType", None) or pltpu.DeviceIdType
_CompilerParams = getattr(pltpu, "CompilerParams", None) or pltpu.TPUCompilerParams


def kernel(x, w_mat, scale_x, scale_w):
    M, k_per = x.shape
    k_per2, N = w_mat.shape
    assert k_per == k_per2
    m_per = M // N_DEV
    N2 = N // 2

    def body(x_ref, w_ref, sx_ref, sw_ref, out_ref,
             xg_ref, wgf_ref, wgb_ref,
             x_send_sems, x_recv_sems,
             f_send_sems, f_recv_sems, b_send_sems, b_recv_sems,
             creditf, creditb):
        my = lax.axis_index("i")
        left = lax.rem(my + N_DEV - 1, N_DEV)
        right = lax.rem(my + 1, N_DEV)

        barrier = pltpu.get_barrier_semaphore()
        for j in range(1, N_DEV):
            _sem_signal(barrier, inc=1,
                        device_id=(lax.rem(my + j, N_DEV),),
                        device_id_type=_DevIdType.MESH)
        _sem_wait(barrier, N_DEV - 1)

        def xg_wait(c):
            xr = pltpu.make_async_remote_copy(
                src_ref=xg_ref.at[c],
                dst_ref=xg_ref.at[c],
                send_sem=x_send_sems.at[c],
                recv_sem=x_recv_sems.at[c],
                device_id=(left,),
                device_id_type=_DevIdType.MESH,
            )
            xr.wait_recv()

        def send(direction, h):
            if direction == 0:
                src = w_ref.at[:, pl.ds(0, N2)] if h == 0 \
                    else wgf_ref.at[(h - 1) % N_SLOT]
                rd = pltpu.make_async_remote_copy(
                    src_ref=src,
                    dst_ref=wgf_ref.at[h % N_SLOT],
                    send_sem=f_send_sems.at[h % 2],
                    recv_sem=f_recv_sems.at[h % 2],
                    device_id=(right,),
                    device_id_type=_DevIdType.MESH,
                )
            else:
                src = w_ref.at[:, pl.ds(N2, N2)] if h == 0 \
                    else wgb_ref.at[(h - 1) % N_SLOT]
                rd = pltpu.make_async_remote_copy(
                    src_ref=src,
                    dst_ref=wgb_ref.at[h % N_SLOT],
                    send_sem=b_send_sems.at[h % 2],
                    recv_sem=b_recv_sems.at[h % 2],
                    device_id=(left,),
                    device_id_type=_DevIdType.MESH,
                )
            rd.start()
            return rd

        bf16 = jnp.bfloat16
        f32 = jnp.float32

        rdf = send(0, 0)
        rdb = send(1, 0)

        x_sends = []
        for j in range(1, N_DEV):
            t = lax.rem(my + j, N_DEV)
            rd = pltpu.make_async_remote_copy(
                src_ref=x_ref.at[pl.ds(t * m_per, m_per), :],
                dst_ref=xg_ref.at[my],
                send_sem=x_send_sems.at[t],
                recv_sem=x_recv_sems.at[my],
                device_id=(t,),
                device_id_type=_DevIdType.MESH,
            )
            rd.start()
            x_sends.append(rd)

        x_own = x_ref[pl.ds(my * m_per, m_per), :]
        out_ref[...] = jnp.dot(
            x_own.astype(bf16), w_ref[...].astype(bf16),
            preferred_element_type=f32)
        rdf.wait_send()
        rdb.wait_send()
        rdf.wait_recv()
        rdb.wait_recv()

        for h in range(1, N_DEV - 1):
            if h >= N_SLOT:
                _sem_wait(creditf.at[h % N_SLOT], 1)
                _sem_wait(creditb.at[h % N_SLOT], 1)
            rdf = send(0, h)
            rdb = send(1, h)

            cf = lax.rem(my + N_DEV - h, N_DEV)
            cb = lax.rem(my + h, N_DEV)
            if h <= 3:
                xg_wait(cf)
                xg_wait(cb)
            elif h == 4:
                xg_wait(cb)
            out_ref[:, pl.ds(0, N2)] += jnp.dot(
                xg_ref[cf].astype(bf16), wgf_ref[(h - 1) % N_SLOT].astype(bf16),
                preferred_element_type=f32)
            out_ref[:, pl.ds(N2, N2)] += jnp.dot(
                xg_ref[cb].astype(bf16), wgb_ref[(h - 1) % N_SLOT].astype(bf16),
                preferred_element_type=f32)

            rdf.wait_send()
            rdb.wait_send()
            if h < N_SLOT:
                _sem_signal(creditf.at[(h - 1) % N_SLOT], inc=1,
                            device_id=(left,),
                            device_id_type=_DevIdType.MESH)
                _sem_signal(creditb.at[(h - 1) % N_SLOT], inc=1,
                            device_id=(right,),
                            device_id_type=_DevIdType.MESH)
            rdf.wait_recv()
            rdb.wait_recv()

        s = sx_ref[0] * sw_ref[0]
        cf = lax.rem(my + 1, N_DEV)
        cb = lax.rem(my + N_DEV - 1, N_DEV)
        last = (N_DEV - 2) % N_SLOT
        out_ref[:, pl.ds(0, N2)] = s * (
            out_ref[:, pl.ds(0, N2)] + jnp.dot(
                xg_ref[cf].astype(bf16), wgf_ref[last].astype(bf16),
                preferred_element_type=f32))
        out_ref[:, pl.ds(N2, N2)] = s * (
            out_ref[:, pl.ds(N2, N2)] + jnp.dot(
                xg_ref[cb].astype(bf16), wgb_ref[last].astype(bf16),
                preferred_element_type=f32))
        for rd in x_sends:
            rd.wait_send()

    return pl.pallas_call(
        body,
        out_shape=jax.ShapeDtypeStruct((m_per, N), jnp.float32),
        in_specs=[
            pl.BlockSpec(memory_space=pltpu.VMEM),
            pl.BlockSpec(memory_space=pltpu.VMEM),
            pl.BlockSpec(memory_space=pltpu.SMEM),
            pl.BlockSpec(memory_space=pltpu.SMEM),
        ],
        out_specs=pl.BlockSpec(memory_space=pltpu.VMEM),
        scratch_shapes=[
            pltpu.VMEM((N_DEV, m_per, k_per), jnp.int8),
            pltpu.VMEM((N_SLOT, k_per, N2), jnp.int8),
            pltpu.VMEM((N_SLOT, k_per, N2), jnp.int8),
            pltpu.SemaphoreType.DMA((N_DEV,)),
            pltpu.SemaphoreType.DMA((N_DEV,)),
            pltpu.SemaphoreType.DMA((2,)),
            pltpu.SemaphoreType.DMA((2,)),
            pltpu.SemaphoreType.DMA((2,)),
            pltpu.SemaphoreType.DMA((2,)),
            pltpu.SemaphoreType.REGULAR((N_SLOT,)),
            pltpu.SemaphoreType.REGULAR((N_SLOT,)),
        ],
        compiler_params=_CompilerParams(
            collective_id=0, vmem_limit_bytes=60 * 1024 * 1024),
    )(x, w_mat, scale_x, scale_w)


# device time: 215022 ns/iter; 1.8687x vs baseline; 1.0005x over previous
import jax
import jax.numpy as jnp
from jax import lax
from jax.experimental import pallas as pl
from jax.experimental.pallas import tpu as pltpu

N_DEV = 8
N_SLOT = 4

_sem_signal = getattr(pl, "semaphore_signal", None) or pltpu.semaphore_signal
_sem_wait = getattr(pl, "semaphore_wait", None) or pltpu.semaphore_wait
_DevIdType = getattr(pl, "DeviceIdType", None) or pltpu.DeviceIdType
_CompilerParams = getattr(pltpu, "CompilerParams", None) or pltpu.TPUCompilerParams


def kernel(x, w_mat, scale_x, scale_w):
    M, k_per = x.shape
    k_per2, N = w_mat.shape
    assert k_per == k_per2
    m_per = M // N_DEV
    N2 = N // 2

    def body(x_ref, w_ref, sx_ref, sw_ref, out_ref,
             xg_ref, wgf_ref, wgb_ref,
             x_send_sems, x_recv_sems,
             f_send_sems, f_recv_sems, b_send_sems, b_recv_sems,
             creditf, creditb):
        my = lax.axis_index("i")
        left = lax.rem(my + N_DEV - 1, N_DEV)
        right = lax.rem(my + 1, N_DEV)

        barrier = pltpu.get_barrier_semaphore()
        for j in range(1, N_DEV):
            _sem_signal(barrier, inc=1,
                        device_id=(lax.rem(my + j, N_DEV),),
                        device_id_type=_DevIdType.MESH)
        _sem_wait(barrier, N_DEV - 1)

        def xg_wait(c):
            xr = pltpu.make_async_remote_copy(
                src_ref=xg_ref.at[c],
                dst_ref=xg_ref.at[c],
                send_sem=x_send_sems.at[c],
                recv_sem=x_recv_sems.at[c],
                device_id=(left,),
                device_id_type=_DevIdType.MESH,
            )
            xr.wait_recv()

        def send(direction, h):
            if direction == 0:
                src = w_ref.at[:, pl.ds(0, N2)] if h == 0 \
                    else wgf_ref.at[(h - 1) % N_SLOT]
                rd = pltpu.make_async_remote_copy(
                    src_ref=src,
                    dst_ref=wgf_ref.at[h % N_SLOT],
                    send_sem=f_send_sems.at[h % 2],
                    recv_sem=f_recv_sems.at[h % 2],
                    device_id=(right,),
                    device_id_type=_DevIdType.MESH,
                )
            else:
                src = w_ref.at[:, pl.ds(N2, N2)] if h == 0 \
                    else wgb_ref.at[(h - 1) % N_SLOT]
                rd = pltpu.make_async_remote_copy(
                    src_ref=src,
                    dst_ref=wgb_ref.at[h % N_SLOT],
                    send_sem=b_send_sems.at[h % 2],
                    recv_sem=b_recv_sems.at[h % 2],
                    device_id=(left,),
                    device_id_type=_DevIdType.MESH,
                )
            rd.start()
            return rd

        bf16 = jnp.bfloat16
        f32 = jnp.float32

        rdf = send(0, 0)
        rdb = send(1, 0)

        x_sends = []
        for j in range(1, N_DEV):
            t = lax.rem(my + j, N_DEV)
            rd = pltpu.make_async_remote_copy(
                src_ref=x_ref.at[pl.ds(t * m_per, m_per), :],
                dst_ref=xg_ref.at[my],
                send_sem=x_send_sems.at[t],
                recv_sem=x_recv_sems.at[my],
                device_id=(t,),
                device_id_type=_DevIdType.MESH,
            )
            rd.start()
            x_sends.append(rd)

        x_own = x_ref[pl.ds(my * m_per, m_per), :]
        out_ref[...] = jnp.dot(
            x_own.astype(bf16), w_ref[...].astype(bf16),
            preferred_element_type=f32)
        rdf.wait_send()
        rdb.wait_send()
        rdf.wait_recv()
        rdb.wait_recv()

        for h in range(1, N_DEV - 1):
            if h >= N_SLOT:
                _sem_wait(creditf.at[h % N_SLOT], 1)
                _sem_wait(creditb.at[h % N_SLOT], 1)
            rdf = send(0, h)
            rdb = send(1, h)

            if h % 2 == 0:
                cf0 = lax.rem(my + N_DEV - h + 1, N_DEV)
                cf1 = lax.rem(my + N_DEV - h, N_DEV)
                cb0 = lax.rem(my + h - 1, N_DEV)
                cb1 = lax.rem(my + h, N_DEV)
                if h == 2:
                    for c in (cf0, cf1, cb0, cb1):
                        xg_wait(c)
                elif h == 4:
                    xg_wait(cf0)
                    xg_wait(cb0)
                    xg_wait(cb1)
                s0 = (h - 2) % N_SLOT
                s1 = (h - 1) % N_SLOT
                xf = jnp.concatenate([xg_ref[cf0], xg_ref[cf1]], axis=1)
                wf = jnp.concatenate([wgf_ref[s0], wgf_ref[s1]], axis=0)
                out_ref[:, pl.ds(0, N2)] += jnp.dot(
                    xf.astype(bf16), wf.astype(bf16),
                    preferred_element_type=f32)
                xb = jnp.concatenate([xg_ref[cb0], xg_ref[cb1]], axis=1)
                wb = jnp.concatenate([wgb_ref[s0], wgb_ref[s1]], axis=0)
                out_ref[:, pl.ds(N2, N2)] += jnp.dot(
                    xb.astype(bf16), wb.astype(bf16),
                    preferred_element_type=f32)

            rdf.wait_send()
            rdb.wait_send()
            if h in (2, 4):
                for slot in ([0, 1] if h == 2 else [2]):
                    _sem_signal(creditf.at[slot], inc=1,
                                device_id=(left,),
                                device_id_type=_DevIdType.MESH)
                    _sem_signal(creditb.at[slot], inc=1,
                                device_id=(right,),
                                device_id_type=_DevIdType.MESH)
            rdf.wait_recv()
            rdb.wait_recv()

        s = sx_ref[0] * sw_ref[0]
        cf = lax.rem(my + 1, N_DEV)
        cb = lax.rem(my + N_DEV - 1, N_DEV)
        last = (N_DEV - 2) % N_SLOT
        out_ref[:, pl.ds(0, N2)] = s * (
            out_ref[:, pl.ds(0, N2)] + jnp.dot(
                xg_ref[cf].astype(bf16), wgf_ref[last].astype(bf16),
                preferred_element_type=f32))
        out_ref[:, pl.ds(N2, N2)] = s * (
            out_ref[:, pl.ds(N2, N2)] + jnp.dot(
                xg_ref[cb].astype(bf16), wgb_ref[last].astype(bf16),
                preferred_element_type=f32))
        for rd in x_sends:
            rd.wait_send()

    return pl.pallas_call(
        body,
        out_shape=jax.ShapeDtypeStruct((m_per, N), jnp.float32),
        in_specs=[
            pl.BlockSpec(memory_space=pltpu.VMEM),
            pl.BlockSpec(memory_space=pltpu.VMEM),
            pl.BlockSpec(memory_space=pltpu.SMEM),
            pl.BlockSpec(memory_space=pltpu.SMEM),
        ],
        out_specs=pl.BlockSpec(memory_space=pltpu.VMEM),
        scratch_shapes=[
            pltpu.VMEM((N_DEV, m_per, k_per), jnp.int8),
            pltpu.VMEM((N_SLOT, k_per, N2), jnp.int8),
            pltpu.VMEM((N_SLOT, k_per, N2), jnp.int8),
            pltpu.SemaphoreType.DMA((N_DEV,)),
            pltpu.SemaphoreType.DMA((N_DEV,)),
            pltpu.SemaphoreType.DMA((2,)),
            pltpu.SemaphoreType.DMA((2,)),
            pltpu.SemaphoreType.DMA((2,)),
            pltpu.SemaphoreType.DMA((2,)),
            pltpu.SemaphoreType.REGULAR((N_SLOT,)),
            pltpu.SemaphoreType.REGULAR((N_SLOT,)),
        ],
        compiler_params=_CompilerParams(
            collective_id=0, vmem_limit_bytes=60 * 1024 * 1024),
    )(x, w_mat, scale_x, scale_w)


# device time: 204121 ns/iter; 1.9685x vs baseline; 1.0534x over previous
import jax
import jax.numpy as jnp
from jax import lax
from jax.experimental import pallas as pl
from jax.experimental.pallas import tpu as pltpu

N_DEV = 8
N_SLOT = 4
N_SUB = 2

_sem_signal = getattr(pl, "semaphore_signal", None) or pltpu.semaphore_signal
_sem_wait = getattr(pl, "semaphore_wait", None) or pltpu.semaphore_wait
_DevIdType = getattr(pl, "DeviceIdType", None) or pltpu.DeviceIdType
_CompilerParams = getattr(pltpu, "CompilerParams", None) or pltpu.TPUCompilerParams


def kernel(x, w_mat, scale_x, scale_w):
    M, k_per = x.shape
    k_per2, N = w_mat.shape
    assert k_per == k_per2
    m_per = M // N_DEV
    N2 = N // 2
    N4 = N2 // N_SUB

    def body(x_ref, w_ref, sx_ref, sw_ref, out_ref,
             xg_ref, wgf_ref, wgb_ref,
             x_send_sems, x_recv_sems,
             f_send_sems, f_recv_sems, b_send_sems, b_recv_sems,
             creditf, creditb):
        my = lax.axis_index("i")
        left = lax.rem(my + N_DEV - 1, N_DEV)
        right = lax.rem(my + 1, N_DEV)

        barrier = pltpu.get_barrier_semaphore()
        for j in range(1, N_DEV):
            _sem_signal(barrier, inc=1,
                        device_id=(lax.rem(my + j, N_DEV),),
                        device_id_type=_DevIdType.MESH)
        _sem_wait(barrier, N_DEV - 1)

        def xg_wait(c):
            xr = pltpu.make_async_remote_copy(
                src_ref=xg_ref.at[c],
                dst_ref=xg_ref.at[c],
                send_sem=x_send_sems.at[c],
                recv_sem=x_recv_sems.at[c],
                device_id=(left,),
                device_id_type=_DevIdType.MESH,
            )
            xr.wait_recv()

        def send(direction, h, sub):
            cs = pl.ds(sub * N4, N4)
            if direction == 0:
                src = w_ref.at[:, pl.ds(sub * N4, N4)] if h == 0 \
                    else wgf_ref.at[(h - 1) % N_SLOT, :, cs]
                rd = pltpu.make_async_remote_copy(
                    src_ref=src,
                    dst_ref=wgf_ref.at[h % N_SLOT, :, cs],
                    send_sem=f_send_sems.at[sub, h % 2],
                    recv_sem=f_recv_sems.at[sub, h % 2],
                    device_id=(right,),
                    device_id_type=_DevIdType.MESH,
                )
            else:
                src = w_ref.at[:, pl.ds(N2 + sub * N4, N4)] if h == 0 \
                    else wgb_ref.at[(h - 1) % N_SLOT, :, cs]
                rd = pltpu.make_async_remote_copy(
                    src_ref=src,
                    dst_ref=wgb_ref.at[h % N_SLOT, :, cs],
                    send_sem=b_send_sems.at[sub, h % 2],
                    recv_sem=b_recv_sems.at[sub, h % 2],
                    device_id=(left,),
                    device_id_type=_DevIdType.MESH,
                )
            rd.start()
            return rd

        bf16 = jnp.bfloat16
        f32 = jnp.float32

        prev = [send(0, 0, 0), send(1, 0, 0), send(0, 0, 1), send(1, 0, 1)]

        x_sends = []
        for j in range(1, N_DEV):
            t = lax.rem(my + j, N_DEV)
            rd = pltpu.make_async_remote_copy(
                src_ref=x_ref.at[pl.ds(t * m_per, m_per), :],
                dst_ref=xg_ref.at[my],
                send_sem=x_send_sems.at[t],
                recv_sem=x_recv_sems.at[my],
                device_id=(t,),
                device_id_type=_DevIdType.MESH,
            )
            rd.start()
            x_sends.append(rd)

        x_own = x_ref[pl.ds(my * m_per, m_per), :]
        out_ref[...] = jnp.dot(
            x_own.astype(bf16), w_ref[...].astype(bf16),
            preferred_element_type=f32)

        for h in range(1, N_DEV - 1):
            if h >= N_SLOT:
                _sem_wait(creditf.at[h % N_SLOT], 1)
                _sem_wait(creditb.at[h % N_SLOT], 1)

            cur = []
            for sub in range(N_SUB):
                prev[2 * sub].wait_recv()
                cur.append(send(0, h, sub))
                prev[2 * sub + 1].wait_recv()
                cur.append(send(1, h, sub))

            if h % 2 == 0:
                cf0 = lax.rem(my + N_DEV - h + 1, N_DEV)
                cf1 = lax.rem(my + N_DEV - h, N_DEV)
                cb0 = lax.rem(my + h - 1, N_DEV)
                cb1 = lax.rem(my + h, N_DEV)
                if h == 2:
                    for c in (cf0, cf1, cb0, cb1):
                        xg_wait(c)
                elif h == 4:
                    xg_wait(cf0)
                    xg_wait(cb0)
                    xg_wait(cb1)
                s0 = (h - 2) % N_SLOT
                s1 = (h - 1) % N_SLOT
                xf = jnp.concatenate([xg_ref[cf0], xg_ref[cf1]], axis=1)
                wf = jnp.concatenate([wgf_ref[s0], wgf_ref[s1]], axis=0)
                out_ref[:, pl.ds(0, N2)] += jnp.dot(
                    xf.astype(bf16), wf.astype(bf16),
                    preferred_element_type=f32)
                xb = jnp.concatenate([xg_ref[cb0], xg_ref[cb1]], axis=1)
                wb = jnp.concatenate([wgb_ref[s0], wgb_ref[s1]], axis=0)
                out_ref[:, pl.ds(N2, N2)] += jnp.dot(
                    xb.astype(bf16), wb.astype(bf16),
                    preferred_element_type=f32)

            for rd in prev:
                rd.wait_send()
            if h in (2, 3, 4):
                _sem_signal(creditf.at[h - 2], inc=1,
                            device_id=(left,),
                            device_id_type=_DevIdType.MESH)
                _sem_signal(creditb.at[h - 2], inc=1,
                            device_id=(right,),
                            device_id_type=_DevIdType.MESH)
            prev = cur

        for rd in prev:
            rd.wait_recv()
        s = sx_ref[0] * sw_ref[0]
        cf = lax.rem(my + 1, N_DEV)
        cb = lax.rem(my + N_DEV - 1, N_DEV)
        last = (N_DEV - 2) % N_SLOT
        out_ref[:, pl.ds(0, N2)] = s * (
            out_ref[:, pl.ds(0, N2)] + jnp.dot(
                xg_ref[cf].astype(bf16), wgf_ref[last].astype(bf16),
                preferred_element_type=f32))
        out_ref[:, pl.ds(N2, N2)] = s * (
            out_ref[:, pl.ds(N2, N2)] + jnp.dot(
                xg_ref[cb].astype(bf16), wgb_ref[last].astype(bf16),
                preferred_element_type=f32))
        for rd in prev:
            rd.wait_send()
        for rd in x_sends:
            rd.wait_send()

    return pl.pallas_call(
        body,
        out_shape=jax.ShapeDtypeStruct((m_per, N), jnp.float32),
        in_specs=[
            pl.BlockSpec(memory_space=pltpu.VMEM),
            pl.BlockSpec(memory_space=pltpu.VMEM),
            pl.BlockSpec(memory_space=pltpu.SMEM),
            pl.BlockSpec(memory_space=pltpu.SMEM),
        ],
        out_specs=pl.BlockSpec(memory_space=pltpu.VMEM),
        scratch_shapes=[
            pltpu.VMEM((N_DEV, m_per, k_per), jnp.int8),
            pltpu.VMEM((N_SLOT, k_per, N2), jnp.int8),
            pltpu.VMEM((N_SLOT, k_per, N2), jnp.int8),
            pltpu.SemaphoreType.DMA((N_DEV,)),
            pltpu.SemaphoreType.DMA((N_DEV,)),
            pltpu.SemaphoreType.DMA((N_SUB, 2)),
            pltpu.SemaphoreType.DMA((N_SUB, 2)),
            pltpu.SemaphoreType.DMA((N_SUB, 2)),
            pltpu.SemaphoreType.DMA((N_SUB, 2)),
            pltpu.SemaphoreType.REGULAR((N_SLOT,)),
            pltpu.SemaphoreType.REGULAR((N_SLOT,)),
        ],
        compiler_params=_CompilerParams(
            collective_id=0, vmem_limit_bytes=60 * 1024 * 1024),
    )(x, w_mat, scale_x, scale_w)
